# baseline (speedup 1.0000x reference)
import functools
import math

import numpy as np
import jax
import jax.numpy as jnp
from jax.experimental import pallas as pl

_H = 16


def _bdot(a, b):
    return jax.lax.dot_general(
        a.astype(jnp.bfloat16), b.astype(jnp.bfloat16),
        (((1,), (0,)), ((), ())), preferred_element_type=jnp.float32)


def _split_bf16(x):
    hi = x.astype(jnp.bfloat16)
    lo = (x - hi.astype(jnp.float32)).astype(jnp.bfloat16)
    return hi, lo


def _dot3(a, bh, bl):
    ah, al = _split_bf16(a)
    return _bdot(ah, bh) + (_bdot(ah, bl) + _bdot(al, bh))


def _dft_mats(L):
    F = L // 2 + 1
    FP = ((F + 127) // 128) * 128
    t = np.arange(L, dtype=np.int64)[:, None]
    f = np.arange(F, dtype=np.int64)[None, :]
    ang = 2.0 * np.pi * ((t * f) % L).astype(np.float64) / L
    CF = np.zeros((L, FP), np.float32)
    SF = np.zeros((L, FP), np.float32)
    CF[:, :F] = np.cos(ang)
    SF[:, :F] = np.sin(ang)
    alpha = np.full((F,), 2.0, np.float64)
    alpha[0] = 1.0
    if L % 2 == 0:
        alpha[F - 1] = 1.0
    ICc = np.zeros((FP, L), np.float32)
    ICs = np.zeros((FP, L), np.float32)
    ICc[:F, :] = (alpha[:, None] / L) * np.cos(ang.T)
    ICs[:F, :] = -(alpha[:, None] / L) * np.sin(ang.T)

    def split(m):
        hi = m.astype(np.dtype(jnp.bfloat16))
        lo = (m - hi.astype(np.float32)).astype(np.dtype(jnp.bfloat16))
        return jnp.asarray(hi), jnp.asarray(lo)

    return split(CF), split(SF), split(ICc), split(ICs)


def _proj_T_kernel(x_ref, w_ref, b_ref, o_ref):
    y = _bdot(x_ref[...], w_ref[...]) + b_ref[...]
    o_ref[...] = y.T


def _proj_to_lanes(x, W, b, B, L, tblk):
    M, K = x.shape
    N = W.shape[1]
    TB = L // tblk
    return pl.pallas_call(
        _proj_T_kernel,
        grid=(M // tblk,),
        in_specs=[pl.BlockSpec((tblk, K), lambda i: (i, 0)),
                  pl.BlockSpec((K, N), lambda i: (0, 0)),
                  pl.BlockSpec((1, N), lambda i: (0, 0))],
        out_specs=pl.BlockSpec((N, tblk), lambda i: (i // TB, i % TB)),
        out_shape=jax.ShapeDtypeStruct((B * N, L), jnp.float32),
        interpret=False,
    )(x, W, b.reshape(1, N))


def _mm_from_lanes_kernel(x_ref, w_ref, b_ref, o_ref):
    o_ref[...] = _bdot(x_ref[...].T, w_ref[...]) + b_ref[...]


def _mm_from_lanes(xt, W, b, B, L, tblk):
    D = W.shape[0]
    N = W.shape[1]
    TB = L // tblk
    return pl.pallas_call(
        _mm_from_lanes_kernel,
        grid=(B * TB,),
        in_specs=[pl.BlockSpec((D, tblk), lambda i: (i // TB, i % TB)),
                  pl.BlockSpec((D, N), lambda i: (0, 0)),
                  pl.BlockSpec((1, N), lambda i: (0, 0))],
        out_specs=pl.BlockSpec((tblk, N), lambda i: (i, 0)),
        out_shape=jax.ShapeDtypeStruct((B * L, N), jnp.float32),
        interpret=False,
    )(xt, W, b.reshape(1, N))


def _mega_kernel(qt_ref, kt_ref, cfh_ref, cfl_ref, sfh_ref, sfl_ref,
                 icch_ref, iccl_ref, icsh_ref, icsl_ref, agg_ref, *, L, k):
    cfh, cfl = cfh_ref[...], cfl_ref[...]
    sfh, sfl = sfh_ref[...], sfl_ref[...]

    def fwd(x):
        xh, xl = _split_bf16(x)
        xr = _bdot(xh, cfh) + (_bdot(xh, cfl) + _bdot(xl, cfh))
        xi = _bdot(xh, sfh) + (_bdot(xh, sfl) + _bdot(xl, sfh))
        return xr, xi

    def inv(sr, si):
        return (_dot3(sr, icch_ref[...], iccl_ref[...])
                + _dot3(si, icsh_ref[...], icsl_ref[...]))

    qr, qi = fwd(qt_ref[...])
    kr, ki = fwd(kt_ref[...])
    R = inv(qr * kr + qi * ki, qr * ki - qi * kr)

    vals = R
    iota = jax.lax.broadcasted_iota(jnp.int32, vals.shape, 1)
    ws, ds = [], []
    for _ in range(k):
        m = jnp.max(vals, axis=1, keepdims=True)
        hit = vals == m
        idx = jnp.min(jnp.where(hit, iota, L), axis=1, keepdims=True)
        ws.append(m)
        ds.append(idx)
        vals = jnp.where(iota == idx, -jnp.inf, vals)
    w = jnp.concatenate(ws, axis=1)
    p = jax.nn.softmax(w, axis=1)
    c = jnp.zeros(R.shape, jnp.float32)
    for i in range(k):
        c = c + jnp.where(iota == ds[i], p[:, i:i + 1], 0.0)

    cr, ci = fwd(c)
    agg_ref[...] = inv(qr * cr + qi * ci, qr * ci - qi * cr)


def _mega(qt, kt, CFp, SFp, ICcp, ICsp, k, blk):
    M, L = qt.shape
    FP = CFp[0].shape[1]
    fmat = lambda: pl.BlockSpec((L, FP), lambda i: (0, 0))
    imat = lambda: pl.BlockSpec((FP, L), lambda i: (0, 0))
    rowblk = lambda: pl.BlockSpec((blk, L), lambda i: (i, 0))
    return pl.pallas_call(
        functools.partial(_mega_kernel, L=L, k=k),
        grid=(M // blk,),
        in_specs=[rowblk(), rowblk(), fmat(), fmat(), fmat(), fmat(),
                  imat(), imat(), imat(), imat()],
        out_specs=rowblk(),
        out_shape=jax.ShapeDtypeStruct((M, L), jnp.float32),
        interpret=False,
    )(qt, kt, CFp[0], CFp[1], SFp[0], SFp[1],
      ICcp[0], ICcp[1], ICsp[0], ICsp[1])


def _fwd_kernel(x_ref, cfh_ref, cfl_ref, sfh_ref, sfl_ref, xr_ref, xi_ref):
    xh, xl = _split_bf16(x_ref[...])
    cfh, cfl = cfh_ref[...], cfl_ref[...]
    sfh, sfl = sfh_ref[...], sfl_ref[...]
    xr_ref[...] = _bdot(xh, cfh) + (_bdot(xh, cfl) + _bdot(xl, cfh))
    xi_ref[...] = _bdot(xh, sfh) + (_bdot(xh, sfl) + _bdot(xl, sfh))


def _fwd_fft(x, CFp, SFp, blk):
    M, L = x.shape
    FP = CFp[0].shape[1]
    mat = lambda: pl.BlockSpec((L, FP), lambda i: (0, 0))
    return pl.pallas_call(
        _fwd_kernel,
        grid=(M // blk,),
        in_specs=[pl.BlockSpec((blk, L), lambda i: (i, 0)),
                  mat(), mat(), mat(), mat()],
        out_specs=[pl.BlockSpec((blk, FP), lambda i: (i, 0)),
                   pl.BlockSpec((blk, FP), lambda i: (i, 0))],
        out_shape=[jax.ShapeDtypeStruct((M, FP), jnp.float32),
                   jax.ShapeDtypeStruct((M, FP), jnp.float32)],
        interpret=False,
    )(x, CFp[0], CFp[1], SFp[0], SFp[1])


def _xinv_kernel(ar_ref, ai_ref, br_ref, bi_ref, icch_ref, iccl_ref,
                 icsh_ref, icsl_ref, o_ref):
    ar, ai = ar_ref[...], ai_ref[...]
    br, bi = br_ref[...], bi_ref[...]
    sr = ar * br + ai * bi
    si = ar * bi - ai * br
    o_ref[...] = (_dot3(sr, icch_ref[...], iccl_ref[...])
                  + _dot3(si, icsh_ref[...], icsl_ref[...]))


def _xcorr_inv(Ar, Ai, Br, Bi, ICcp, ICsp, blk):
    M, FP = Ar.shape
    L = ICcp[0].shape[1]
    row = lambda: pl.BlockSpec((blk, FP), lambda i: (i, 0))
    mat = lambda: pl.BlockSpec((FP, L), lambda i: (0, 0))
    return pl.pallas_call(
        _xinv_kernel,
        grid=(M // blk,),
        in_specs=[row(), row(), row(), row(), mat(), mat(), mat(), mat()],
        out_specs=pl.BlockSpec((blk, L), lambda i: (i, 0)),
        out_shape=jax.ShapeDtypeStruct((M, L), jnp.float32),
        interpret=False,
    )(Ar, Ai, Br, Bi, ICcp[0], ICcp[1], ICsp[0], ICsp[1])


def _topk_c_kernel(r_ref, c_ref, *, L, k):
    vals = r_ref[...]
    iota = jax.lax.broadcasted_iota(jnp.int32, vals.shape, 1)
    ws, ds = [], []
    for _ in range(k):
        m = jnp.max(vals, axis=1, keepdims=True)
        hit = vals == m
        idx = jnp.min(jnp.where(hit, iota, L), axis=1, keepdims=True)
        sel = iota == idx
        ws.append(m)
        ds.append(idx)
        vals = jnp.where(sel, -jnp.inf, vals)
    w = jnp.concatenate(ws, axis=1)
    p = jax.nn.softmax(w, axis=1)
    acc = jnp.zeros(r_ref.shape, jnp.float32)
    for i in range(k):
        acc = acc + jnp.where(iota == ds[i], p[:, i:i + 1], 0.0)
    c_ref[...] = acc


def _topk_c(R, k, blk):
    M, L = R.shape
    return pl.pallas_call(
        functools.partial(_topk_c_kernel, L=L, k=k),
        grid=(M // blk,),
        in_specs=[pl.BlockSpec((blk, L), lambda i: (i, 0))],
        out_specs=pl.BlockSpec((blk, L), lambda i: (i, 0)),
        out_shape=jax.ShapeDtypeStruct((M, L), jnp.float32),
        interpret=False,
    )(R)


def _pipeline(q, k, Wq, bq, Wk, bk, Wo, bo):
    B, L, D = q.shape
    H = _H
    depth = D // H
    lanes = B * H * depth
    kk = int(2 * math.log(L))

    mm_blk = min(512, L)
    lane_blk = min(256, lanes)

    qt = _proj_to_lanes(q.reshape(B * L, D), Wq, bq, B, L, mm_blk)
    kt = _proj_to_lanes(k.reshape(B * L, D), Wk, bk, B, L, mm_blk)

    CFp, SFp, ICcp, ICsp = _dft_mats(L)
    Qr, Qi = _fwd_fft(qt, CFp, SFp, lane_blk)
    Kr, Ki = _fwd_fft(kt, CFp, SFp, lane_blk)
    R = _xcorr_inv(Qr, Qi, Kr, Ki, ICcp, ICsp, lane_blk)
    c = _topk_c(R, kk, lane_blk)
    Cr, Ci = _fwd_fft(c, CFp, SFp, lane_blk)
    agg = _xcorr_inv(Qr, Qi, Cr, Ci, ICcp, ICsp, lane_blk)

    out = _mm_from_lanes(agg, Wo, bo, B, L, mm_blk)
    return out.reshape(B, L, D)


def kernel(q, k, v, Wq, bq, Wk, bk, Wv, bv, Wo, bo):
    B = q.shape[0]
    devs = jax.devices()
    nd = 2 if (len(devs) >= 2 and B % 2 == 0) else 1
    if nd == 1:
        return _pipeline(q, k, Wq, bq, Wk, bk, Wo, bo)
    from jax.experimental.shard_map import shard_map
    from jax.sharding import Mesh, PartitionSpec as P
    mesh = Mesh(np.array(devs[:nd]), ("x",))
    fn = shard_map(
        _pipeline, mesh=mesh,
        in_specs=(P("x"), P("x"), P(), P(), P(), P(), P(), P()),
        out_specs=P("x"), check_rep=False)
    return fn(q, k, Wq, bq, Wk, bk, Wo, bo)


# speedup vs baseline: 1.7023x; 1.7023x over previous
import functools
import math

import numpy as np
import jax
import jax.numpy as jnp
from jax.experimental import pallas as pl

_H = 16


def _bdot(a, b):
    return jax.lax.dot_general(
        a.astype(jnp.bfloat16), b.astype(jnp.bfloat16),
        (((1,), (0,)), ((), ())), preferred_element_type=jnp.float32)


def _split_bf16(x):
    hi = x.astype(jnp.bfloat16)
    lo = (x - hi.astype(jnp.float32)).astype(jnp.bfloat16)
    return hi, lo


def _dot3(a, bh, bl):
    ah, al = _split_bf16(a)
    return _bdot(ah, bh) + (_bdot(ah, bl) + _bdot(al, bh))


def _dft_mats(L):
    F = L // 2 + 1
    FP = ((F + 127) // 128) * 128
    t = np.arange(L, dtype=np.int64)[:, None]
    f = np.arange(F, dtype=np.int64)[None, :]
    ang = 2.0 * np.pi * ((t * f) % L).astype(np.float64) / L
    CF = np.zeros((L, FP), np.float32)
    SF = np.zeros((L, FP), np.float32)
    CF[:, :F] = np.cos(ang)
    SF[:, :F] = np.sin(ang)
    alpha = np.full((F,), 2.0, np.float64)
    alpha[0] = 1.0
    if L % 2 == 0:
        alpha[F - 1] = 1.0
    ICc = np.zeros((FP, L), np.float32)
    ICs = np.zeros((FP, L), np.float32)
    ICc[:F, :] = (alpha[:, None] / L) * np.cos(ang.T)
    ICs[:F, :] = -(alpha[:, None] / L) * np.sin(ang.T)

    def split(m):
        hi = m.astype(np.dtype(jnp.bfloat16))
        lo = (m - hi.astype(np.float32)).astype(np.dtype(jnp.bfloat16))
        return jnp.asarray(hi), jnp.asarray(lo)

    return split(CF), split(SF), split(ICc), split(ICs)


def _proj_T_kernel(x_ref, w_ref, b_ref, o_ref):
    y = _bdot(x_ref[...], w_ref[...]) + b_ref[...]
    o_ref[...] = y.T


def _proj_to_lanes(x, W, b, B, L, tblk):
    M, K = x.shape
    N = W.shape[1]
    TB = L // tblk
    return pl.pallas_call(
        _proj_T_kernel,
        grid=(M // tblk,),
        in_specs=[pl.BlockSpec((tblk, K), lambda i: (i, 0)),
                  pl.BlockSpec((K, N), lambda i: (0, 0)),
                  pl.BlockSpec((1, N), lambda i: (0, 0))],
        out_specs=pl.BlockSpec((N, tblk), lambda i: (i // TB, i % TB)),
        out_shape=jax.ShapeDtypeStruct((B * N, L), jnp.float32),
        interpret=False,
    )(x, W, b.reshape(1, N))


def _mm_from_lanes_kernel(x_ref, w_ref, b_ref, o_ref):
    o_ref[...] = _bdot(x_ref[...].T, w_ref[...]) + b_ref[...]


def _mm_from_lanes(xt, W, b, B, L, tblk):
    D = W.shape[0]
    N = W.shape[1]
    TB = L // tblk
    return pl.pallas_call(
        _mm_from_lanes_kernel,
        grid=(B * TB,),
        in_specs=[pl.BlockSpec((D, tblk), lambda i: (i // TB, i % TB)),
                  pl.BlockSpec((D, N), lambda i: (0, 0)),
                  pl.BlockSpec((1, N), lambda i: (0, 0))],
        out_specs=pl.BlockSpec((tblk, N), lambda i: (i, 0)),
        out_shape=jax.ShapeDtypeStruct((B * L, N), jnp.float32),
        interpret=False,
    )(xt, W, b.reshape(1, N))


def _mega_kernel(qt_ref, kt_ref, cfh_ref, cfl_ref, sfh_ref, sfl_ref,
                 icch_ref, iccl_ref, icsh_ref, icsl_ref, agg_ref, *, L, k):
    cfh, cfl = cfh_ref[...], cfl_ref[...]
    sfh, sfl = sfh_ref[...], sfl_ref[...]

    def fwd(x):
        xh, xl = _split_bf16(x)
        xr = _bdot(xh, cfh) + (_bdot(xh, cfl) + _bdot(xl, cfh))
        xi = _bdot(xh, sfh) + (_bdot(xh, sfl) + _bdot(xl, sfh))
        return xr, xi

    def inv(sr, si):
        return (_dot3(sr, icch_ref[...], iccl_ref[...])
                + _dot3(si, icsh_ref[...], icsl_ref[...]))

    qr, qi = fwd(qt_ref[...])
    kr, ki = fwd(kt_ref[...])
    R = inv(qr * kr + qi * ki, qr * ki - qi * kr)

    vals = R
    iota = jax.lax.broadcasted_iota(jnp.int32, vals.shape, 1)
    ws, ds = [], []
    for _ in range(k):
        m = jnp.max(vals, axis=1, keepdims=True)
        hit = vals == m
        idx = jnp.min(jnp.where(hit, iota, L), axis=1, keepdims=True)
        ws.append(m)
        ds.append(idx)
        vals = jnp.where(iota == idx, -jnp.inf, vals)
    w = jnp.concatenate(ws, axis=1)
    p = jax.nn.softmax(w, axis=1)
    c = jnp.zeros(R.shape, jnp.float32)
    for i in range(k):
        c = c + jnp.where(iota == ds[i], p[:, i:i + 1], 0.0)

    cr, ci = fwd(c)
    agg_ref[...] = inv(qr * cr + qi * ci, qr * ci - qi * cr)


def _mega(qt, kt, CFp, SFp, ICcp, ICsp, k, blk):
    M, L = qt.shape
    FP = CFp[0].shape[1]
    fmat = lambda: pl.BlockSpec((L, FP), lambda i: (0, 0))
    imat = lambda: pl.BlockSpec((FP, L), lambda i: (0, 0))
    rowblk = lambda: pl.BlockSpec((blk, L), lambda i: (i, 0))
    return pl.pallas_call(
        functools.partial(_mega_kernel, L=L, k=k),
        grid=(M // blk,),
        in_specs=[rowblk(), rowblk(), fmat(), fmat(), fmat(), fmat(),
                  imat(), imat(), imat(), imat()],
        out_specs=rowblk(),
        out_shape=jax.ShapeDtypeStruct((M, L), jnp.float32),
        interpret=False,
    )(qt, kt, CFp[0], CFp[1], SFp[0], SFp[1],
      ICcp[0], ICcp[1], ICsp[0], ICsp[1])


def _fwd_kernel(x_ref, cfh_ref, cfl_ref, sfh_ref, sfl_ref, xr_ref, xi_ref):
    xh, xl = _split_bf16(x_ref[...])
    cfh, cfl = cfh_ref[...], cfl_ref[...]
    sfh, sfl = sfh_ref[...], sfl_ref[...]
    xr_ref[...] = _bdot(xh, cfh) + (_bdot(xh, cfl) + _bdot(xl, cfh))
    xi_ref[...] = _bdot(xh, sfh) + (_bdot(xh, sfl) + _bdot(xl, sfh))


def _fwd_fft(x, CFp, SFp, blk):
    M, L = x.shape
    FP = CFp[0].shape[1]
    mat = lambda: pl.BlockSpec((L, FP), lambda i: (0, 0))
    return pl.pallas_call(
        _fwd_kernel,
        grid=(M // blk,),
        in_specs=[pl.BlockSpec((blk, L), lambda i: (i, 0)),
                  mat(), mat(), mat(), mat()],
        out_specs=[pl.BlockSpec((blk, FP), lambda i: (i, 0)),
                   pl.BlockSpec((blk, FP), lambda i: (i, 0))],
        out_shape=[jax.ShapeDtypeStruct((M, FP), jnp.float32),
                   jax.ShapeDtypeStruct((M, FP), jnp.float32)],
        interpret=False,
    )(x, CFp[0], CFp[1], SFp[0], SFp[1])


def _xinv_kernel(ar_ref, ai_ref, br_ref, bi_ref, icch_ref, iccl_ref,
                 icsh_ref, icsl_ref, o_ref):
    ar, ai = ar_ref[...], ai_ref[...]
    br, bi = br_ref[...], bi_ref[...]
    sr = ar * br + ai * bi
    si = ar * bi - ai * br
    o_ref[...] = (_dot3(sr, icch_ref[...], iccl_ref[...])
                  + _dot3(si, icsh_ref[...], icsl_ref[...]))


def _xcorr_inv(Ar, Ai, Br, Bi, ICcp, ICsp, blk):
    M, FP = Ar.shape
    L = ICcp[0].shape[1]
    row = lambda: pl.BlockSpec((blk, FP), lambda i: (i, 0))
    mat = lambda: pl.BlockSpec((FP, L), lambda i: (0, 0))
    return pl.pallas_call(
        _xinv_kernel,
        grid=(M // blk,),
        in_specs=[row(), row(), row(), row(), mat(), mat(), mat(), mat()],
        out_specs=pl.BlockSpec((blk, L), lambda i: (i, 0)),
        out_shape=jax.ShapeDtypeStruct((M, L), jnp.float32),
        interpret=False,
    )(Ar, Ai, Br, Bi, ICcp[0], ICcp[1], ICsp[0], ICsp[1])


def _topk_c_kernel(r_ref, c_ref, *, L, k):
    vals = r_ref[...]
    iota = jax.lax.broadcasted_iota(jnp.int32, vals.shape, 1)
    ws, ds = [], []
    for _ in range(k):
        m = jnp.max(vals, axis=1, keepdims=True)
        hit = vals == m
        idx = jnp.min(jnp.where(hit, iota, L), axis=1, keepdims=True)
        sel = iota == idx
        ws.append(m)
        ds.append(idx)
        vals = jnp.where(sel, -jnp.inf, vals)
    w = jnp.concatenate(ws, axis=1)
    p = jax.nn.softmax(w, axis=1)
    acc = jnp.zeros(r_ref.shape, jnp.float32)
    for i in range(k):
        acc = acc + jnp.where(iota == ds[i], p[:, i:i + 1], 0.0)
    c_ref[...] = acc


def _topk_c(R, k, blk):
    M, L = R.shape
    return pl.pallas_call(
        functools.partial(_topk_c_kernel, L=L, k=k),
        grid=(M // blk,),
        in_specs=[pl.BlockSpec((blk, L), lambda i: (i, 0))],
        out_specs=pl.BlockSpec((blk, L), lambda i: (i, 0)),
        out_shape=jax.ShapeDtypeStruct((M, L), jnp.float32),
        interpret=False,
    )(R)


def _pipeline(q, k, Wq, bq, Wk, bk, Wo, bo):
    B, L, D = q.shape
    H = _H
    depth = D // H
    lanes = B * H * depth
    kk = int(2 * math.log(L))

    mm_blk = min(512, L)
    lane_blk = min(256, lanes)

    qt = _proj_to_lanes(q.reshape(B * L, D), Wq, bq, B, L, mm_blk)
    kt = _proj_to_lanes(k.reshape(B * L, D), Wk, bk, B, L, mm_blk)

    CFp, SFp, ICcp, ICsp = _dft_mats(L)
    Qr, Qi = _fwd_fft(qt, CFp, SFp, lane_blk)
    Kr, Ki = _fwd_fft(kt, CFp, SFp, lane_blk)
    R = _xcorr_inv(Qr, Qi, Kr, Ki, ICcp, ICsp, lane_blk)
    c = _topk_c(R, kk, lane_blk)
    Cr, Ci = _fwd_fft(c, CFp, SFp, lane_blk)
    agg = _xcorr_inv(Qr, Qi, Cr, Ci, ICcp, ICsp, lane_blk)

    out = _mm_from_lanes(agg, Wo, bo, B, L, mm_blk)
    return out.reshape(B, L, D)


def kernel(q, k, v, Wq, bq, Wk, bk, Wv, bv, Wo, bo):
    B = q.shape[0]
    devs = jax.devices()
    nd = 2 if (len(devs) >= 2 and B % 2 == 0) else 1
    if nd == 1:
        return _pipeline(q, k, Wq, bq, Wk, bk, Wo, bo)
    from jax.experimental.shard_map import shard_map
    from jax.sharding import Mesh, PartitionSpec as P
    mesh = Mesh(np.array(devs[:nd]), ("x",))
    Bloc = B // nd

    def body(q, k, Wq, bq, Wk, bk, Wo, bo):
        i = jax.lax.axis_index("x")
        qloc = jax.lax.dynamic_slice_in_dim(q, i * Bloc, Bloc, 0)
        kloc = jax.lax.dynamic_slice_in_dim(k, i * Bloc, Bloc, 0)
        return _pipeline(qloc, kloc, Wq, bq, Wk, bk, Wo, bo)

    fn = shard_map(
        body, mesh=mesh,
        in_specs=(P(), P(), P(), P(), P(), P(), P(), P()),
        out_specs=P("x"), check_rep=False)
    return fn(q, k, Wq, bq, Wk, bk, Wo, bo)


# speedup vs baseline: 1.7377x; 1.0208x over previous
import functools
import math

import numpy as np
import jax
import jax.numpy as jnp
from jax.experimental import pallas as pl

_H = 16


def _bdot(a, b):
    return jax.lax.dot_general(
        a.astype(jnp.bfloat16), b.astype(jnp.bfloat16),
        (((1,), (0,)), ((), ())), preferred_element_type=jnp.float32)


def _split_bf16(x):
    hi = x.astype(jnp.bfloat16)
    lo = (x - hi.astype(jnp.float32)).astype(jnp.bfloat16)
    return hi, lo


def _dot3(a, bh, bl):
    ah, al = _split_bf16(a)
    return _bdot(ah, bh) + (_bdot(ah, bl) + _bdot(al, bh))


def _dft_mats(L):
    F = L // 2 + 1
    FP = ((F + 127) // 128) * 128
    t = np.arange(L, dtype=np.int64)[:, None]
    f = np.arange(F, dtype=np.int64)[None, :]
    ang = 2.0 * np.pi * ((t * f) % L).astype(np.float64) / L
    CF = np.zeros((L, FP), np.float32)
    SF = np.zeros((L, FP), np.float32)
    CF[:, :F] = np.cos(ang)
    SF[:, :F] = np.sin(ang)
    alpha = np.full((F,), 2.0, np.float64)
    alpha[0] = 1.0
    if L % 2 == 0:
        alpha[F - 1] = 1.0
    ICc = np.zeros((FP, L), np.float32)
    ICs = np.zeros((FP, L), np.float32)
    ICc[:F, :] = (alpha[:, None] / L) * np.cos(ang.T)
    ICs[:F, :] = -(alpha[:, None] / L) * np.sin(ang.T)

    def split(m):
        hi = m.astype(np.dtype(jnp.bfloat16))
        lo = (m - hi.astype(np.float32)).astype(np.dtype(jnp.bfloat16))
        return jnp.asarray(hi), jnp.asarray(lo)

    return split(CF), split(SF), split(ICc), split(ICs)


def _proj_T_kernel(x_ref, w_ref, b_ref, o_ref):
    y = _bdot(x_ref[...], w_ref[...]) + b_ref[...]
    o_ref[...] = y.T


def _proj_to_lanes(x, W, b, B, L, tblk):
    M, K = x.shape
    N = W.shape[1]
    TB = L // tblk
    return pl.pallas_call(
        _proj_T_kernel,
        grid=(M // tblk,),
        in_specs=[pl.BlockSpec((tblk, K), lambda i: (i, 0)),
                  pl.BlockSpec((K, N), lambda i: (0, 0)),
                  pl.BlockSpec((1, N), lambda i: (0, 0))],
        out_specs=pl.BlockSpec((N, tblk), lambda i: (i // TB, i % TB)),
        out_shape=jax.ShapeDtypeStruct((B * N, L), jnp.float32),
        interpret=False,
    )(x, W, b.reshape(1, N))


def _mm_from_lanes_kernel(x_ref, w_ref, b_ref, o_ref):
    o_ref[...] = _bdot(x_ref[...].T, w_ref[...]) + b_ref[...]


def _mm_from_lanes(xt, W, b, B, L, tblk):
    D = W.shape[0]
    N = W.shape[1]
    TB = L // tblk
    return pl.pallas_call(
        _mm_from_lanes_kernel,
        grid=(B * TB,),
        in_specs=[pl.BlockSpec((D, tblk), lambda i: (i // TB, i % TB)),
                  pl.BlockSpec((D, N), lambda i: (0, 0)),
                  pl.BlockSpec((1, N), lambda i: (0, 0))],
        out_specs=pl.BlockSpec((tblk, N), lambda i: (i, 0)),
        out_shape=jax.ShapeDtypeStruct((B * L, N), jnp.float32),
        interpret=False,
    )(xt, W, b.reshape(1, N))


def _fwd2_kernel(qt_ref, kt_ref, cfh_ref, cfl_ref, sfh_ref, sfl_ref,
                 qr_ref, qi_ref, kr_ref, ki_ref):
    cfh, cfl = cfh_ref[...], cfl_ref[...]
    sfh, sfl = sfh_ref[...], sfl_ref[...]

    def fwd(x):
        xh, xl = _split_bf16(x)
        xr = _bdot(xh, cfh) + (_bdot(xh, cfl) + _bdot(xl, cfh))
        xi = _bdot(xh, sfh) + (_bdot(xh, sfl) + _bdot(xl, sfh))
        return xr, xi

    qr_ref[...], qi_ref[...] = fwd(qt_ref[...])
    kr_ref[...], ki_ref[...] = fwd(kt_ref[...])


def _fwd2(qt, kt, CFp, SFp, blk):
    M, L = qt.shape
    FP = CFp[0].shape[1]
    fmat = lambda: pl.BlockSpec((L, FP), lambda i: (0, 0))
    rowL = lambda: pl.BlockSpec((blk, L), lambda i: (i, 0))
    rowF = lambda: pl.BlockSpec((blk, FP), lambda i: (i, 0))
    o = jax.ShapeDtypeStruct((M, FP), jnp.float32)
    return pl.pallas_call(
        _fwd2_kernel,
        grid=(M // blk,),
        in_specs=[rowL(), rowL(), fmat(), fmat(), fmat(), fmat()],
        out_specs=[rowF(), rowF(), rowF(), rowF()],
        out_shape=[o, o, o, o],
        interpret=False,
    )(qt, kt, CFp[0], CFp[1], SFp[0], SFp[1])


def _rsel_kernel(qr_ref, qi_ref, kr_ref, ki_ref, icch_ref, iccl_ref,
                 icsh_ref, icsl_ref, c_ref, *, L, k):
    qr, qi = qr_ref[...], qi_ref[...]
    kr, ki = kr_ref[...], ki_ref[...]
    vals = (_dot3(qr * kr + qi * ki, icch_ref[...], iccl_ref[...])
            + _dot3(qr * ki - qi * kr, icsh_ref[...], icsl_ref[...]))
    iota = jax.lax.broadcasted_iota(jnp.int32, vals.shape, 1)
    ws, ds = [], []
    for _ in range(k):
        m = jnp.max(vals, axis=1, keepdims=True)
        idx = jnp.min(jnp.where(vals == m, iota, L), axis=1, keepdims=True)
        ws.append(m)
        ds.append(idx)
        vals = jnp.where(iota == idx, -jnp.inf, vals)
    w = jnp.concatenate(ws, axis=1)
    p = jax.nn.softmax(w, axis=1)
    c = jnp.zeros((vals.shape[0], L), jnp.float32)
    for i in range(k):
        c = c + jnp.where(iota == ds[i], p[:, i:i + 1], 0.0)
    c_ref[...] = c


def _rsel(Qr, Qi, Kr, Ki, ICcp, ICsp, kk, blk):
    M, FP = Qr.shape
    L = ICcp[0].shape[1]
    imat = lambda: pl.BlockSpec((FP, L), lambda i: (0, 0))
    rowF = lambda: pl.BlockSpec((blk, FP), lambda i: (i, 0))
    return pl.pallas_call(
        functools.partial(_rsel_kernel, L=L, k=kk),
        grid=(M // blk,),
        in_specs=[rowF(), rowF(), rowF(), rowF(),
                  imat(), imat(), imat(), imat()],
        out_specs=pl.BlockSpec((blk, L), lambda i: (i, 0)),
        out_shape=jax.ShapeDtypeStruct((M, L), jnp.float32),
        interpret=False,
    )(Qr, Qi, Kr, Ki, ICcp[0], ICcp[1], ICsp[0], ICsp[1])


def _recon_kernel(c_ref, qr_ref, qi_ref, cfh_ref, cfl_ref, sfh_ref, sfl_ref,
                  icch_ref, iccl_ref, icsh_ref, icsl_ref, agg_ref):
    ch, cl = _split_bf16(c_ref[...])
    cfh, cfl = cfh_ref[...], cfl_ref[...]
    sfh, sfl = sfh_ref[...], sfl_ref[...]
    cr = _bdot(ch, cfh) + (_bdot(ch, cfl) + _bdot(cl, cfh))
    ci = _bdot(ch, sfh) + (_bdot(ch, sfl) + _bdot(cl, sfh))
    qr, qi = qr_ref[...], qi_ref[...]
    agg_ref[...] = (_dot3(qr * cr + qi * ci, icch_ref[...], iccl_ref[...])
                    + _dot3(qr * ci - qi * cr, icsh_ref[...], icsl_ref[...]))


def _recon(c, Qr, Qi, CFp, SFp, ICcp, ICsp, blk):
    M, L = c.shape
    FP = CFp[0].shape[1]
    fmat = lambda: pl.BlockSpec((L, FP), lambda i: (0, 0))
    imat = lambda: pl.BlockSpec((FP, L), lambda i: (0, 0))
    rowL = lambda: pl.BlockSpec((blk, L), lambda i: (i, 0))
    rowF = lambda: pl.BlockSpec((blk, FP), lambda i: (i, 0))
    return pl.pallas_call(
        _recon_kernel,
        grid=(M // blk,),
        in_specs=[rowL(), rowF(), rowF(), fmat(), fmat(), fmat(), fmat(),
                  imat(), imat(), imat(), imat()],
        out_specs=rowL(),
        out_shape=jax.ShapeDtypeStruct((M, L), jnp.float32),
        interpret=False,
    )(c, Qr, Qi, CFp[0], CFp[1], SFp[0], SFp[1],
      ICcp[0], ICcp[1], ICsp[0], ICsp[1])


def _fwd_kernel(x_ref, cfh_ref, cfl_ref, sfh_ref, sfl_ref, xr_ref, xi_ref):
    xh, xl = _split_bf16(x_ref[...])
    cfh, cfl = cfh_ref[...], cfl_ref[...]
    sfh, sfl = sfh_ref[...], sfl_ref[...]
    xr_ref[...] = _bdot(xh, cfh) + (_bdot(xh, cfl) + _bdot(xl, cfh))
    xi_ref[...] = _bdot(xh, sfh) + (_bdot(xh, sfl) + _bdot(xl, sfh))


def _fwd_fft(x, CFp, SFp, blk):
    M, L = x.shape
    FP = CFp[0].shape[1]
    mat = lambda: pl.BlockSpec((L, FP), lambda i: (0, 0))
    return pl.pallas_call(
        _fwd_kernel,
        grid=(M // blk,),
        in_specs=[pl.BlockSpec((blk, L), lambda i: (i, 0)),
                  mat(), mat(), mat(), mat()],
        out_specs=[pl.BlockSpec((blk, FP), lambda i: (i, 0)),
                   pl.BlockSpec((blk, FP), lambda i: (i, 0))],
        out_shape=[jax.ShapeDtypeStruct((M, FP), jnp.float32),
                   jax.ShapeDtypeStruct((M, FP), jnp.float32)],
        interpret=False,
    )(x, CFp[0], CFp[1], SFp[0], SFp[1])


def _xinv_kernel(ar_ref, ai_ref, br_ref, bi_ref, icch_ref, iccl_ref,
                 icsh_ref, icsl_ref, o_ref):
    ar, ai = ar_ref[...], ai_ref[...]
    br, bi = br_ref[...], bi_ref[...]
    sr = ar * br + ai * bi
    si = ar * bi - ai * br
    o_ref[...] = (_dot3(sr, icch_ref[...], iccl_ref[...])
                  + _dot3(si, icsh_ref[...], icsl_ref[...]))


def _xcorr_inv(Ar, Ai, Br, Bi, ICcp, ICsp, blk):
    M, FP = Ar.shape
    L = ICcp[0].shape[1]
    row = lambda: pl.BlockSpec((blk, FP), lambda i: (i, 0))
    mat = lambda: pl.BlockSpec((FP, L), lambda i: (0, 0))
    return pl.pallas_call(
        _xinv_kernel,
        grid=(M // blk,),
        in_specs=[row(), row(), row(), row(), mat(), mat(), mat(), mat()],
        out_specs=pl.BlockSpec((blk, L), lambda i: (i, 0)),
        out_shape=jax.ShapeDtypeStruct((M, L), jnp.float32),
        interpret=False,
    )(Ar, Ai, Br, Bi, ICcp[0], ICcp[1], ICsp[0], ICsp[1])


def _topk_c_kernel(r_ref, c_ref, *, L, k):
    vals = r_ref[...]
    iota = jax.lax.broadcasted_iota(jnp.int32, vals.shape, 1)
    ws, ds = [], []
    for _ in range(k):
        m = jnp.max(vals, axis=1, keepdims=True)
        hit = vals == m
        idx = jnp.min(jnp.where(hit, iota, L), axis=1, keepdims=True)
        sel = iota == idx
        ws.append(m)
        ds.append(idx)
        vals = jnp.where(sel, -jnp.inf, vals)
    w = jnp.concatenate(ws, axis=1)
    p = jax.nn.softmax(w, axis=1)
    acc = jnp.zeros(r_ref.shape, jnp.float32)
    for i in range(k):
        acc = acc + jnp.where(iota == ds[i], p[:, i:i + 1], 0.0)
    c_ref[...] = acc


def _topk_c(R, k, blk):
    M, L = R.shape
    return pl.pallas_call(
        functools.partial(_topk_c_kernel, L=L, k=k),
        grid=(M // blk,),
        in_specs=[pl.BlockSpec((blk, L), lambda i: (i, 0))],
        out_specs=pl.BlockSpec((blk, L), lambda i: (i, 0)),
        out_shape=jax.ShapeDtypeStruct((M, L), jnp.float32),
        interpret=False,
    )(R)


def _pipeline(q, k, Wq, bq, Wk, bk, Wo, bo):
    B, L, D = q.shape
    H = _H
    depth = D // H
    lanes = B * H * depth
    kk = int(2 * math.log(L))

    mm_blk = min(512, L)
    lane_blk = min(256, lanes)

    qt = _proj_to_lanes(q.reshape(B * L, D), Wq, bq, B, L, mm_blk)
    kt = _proj_to_lanes(k.reshape(B * L, D), Wk, bk, B, L, mm_blk)

    CFp, SFp, ICcp, ICsp = _dft_mats(L)
    Qr, Qi, Kr, Ki = _fwd2(qt, kt, CFp, SFp, lane_blk)
    c = _rsel(Qr, Qi, Kr, Ki, ICcp, ICsp, kk, lane_blk)
    agg = _recon(c, Qr, Qi, CFp, SFp, ICcp, ICsp, lane_blk)

    out = _mm_from_lanes(agg, Wo, bo, B, L, mm_blk)
    return out.reshape(B, L, D)


def kernel(q, k, v, Wq, bq, Wk, bk, Wv, bv, Wo, bo):
    B = q.shape[0]
    devs = jax.devices()
    nd = 2 if (len(devs) >= 2 and B % 2 == 0) else 1
    if nd == 1:
        return _pipeline(q, k, Wq, bq, Wk, bk, Wo, bo)
    from jax.experimental.shard_map import shard_map
    from jax.sharding import Mesh, PartitionSpec as P
    mesh = Mesh(np.array(devs[:nd]), ("x",))
    Bloc = B // nd

    def body(q, k, Wq, bq, Wk, bk, Wo, bo):
        i = jax.lax.axis_index("x")
        qloc = jax.lax.dynamic_slice_in_dim(q, i * Bloc, Bloc, 0)
        kloc = jax.lax.dynamic_slice_in_dim(k, i * Bloc, Bloc, 0)
        return _pipeline(qloc, kloc, Wq, bq, Wk, bk, Wo, bo)

    fn = shard_map(
        body, mesh=mesh,
        in_specs=(P(), P(), P(), P(), P(), P(), P(), P()),
        out_specs=P("x"), check_rep=False)
    return fn(q, k, Wq, bq, Wk, bk, Wo, bo)


# speedup vs baseline: 1.7415x; 1.0022x over previous
import functools
import math

import numpy as np
import jax
import jax.numpy as jnp
from jax.experimental import pallas as pl

_H = 16


def _bdot(a, b):
    return jax.lax.dot_general(
        a.astype(jnp.bfloat16), b.astype(jnp.bfloat16),
        (((1,), (0,)), ((), ())), preferred_element_type=jnp.float32)


def _split_bf16(x):
    hi = x.astype(jnp.bfloat16)
    lo = (x - hi.astype(jnp.float32)).astype(jnp.bfloat16)
    return hi, lo


def _dot3(a, bh, bl):
    ah, al = _split_bf16(a)
    return _bdot(ah, bh) + (_bdot(ah, bl) + _bdot(al, bh))


def _dft_mats(L):
    F = L // 2 + 1
    FP = ((F + 127) // 128) * 128
    t = np.arange(L, dtype=np.int64)[:, None]
    f = np.arange(F, dtype=np.int64)[None, :]
    ang = 2.0 * np.pi * ((t * f) % L).astype(np.float64) / L
    CF = np.zeros((L, FP), np.float32)
    SF = np.zeros((L, FP), np.float32)
    CF[:, :F] = np.cos(ang)
    SF[:, :F] = np.sin(ang)
    alpha = np.full((F,), 2.0, np.float64)
    alpha[0] = 1.0
    if L % 2 == 0:
        alpha[F - 1] = 1.0
    ICc = np.zeros((FP, L), np.float32)
    ICs = np.zeros((FP, L), np.float32)
    ICc[:F, :] = (alpha[:, None] / L) * np.cos(ang.T)
    ICs[:F, :] = -(alpha[:, None] / L) * np.sin(ang.T)

    def split(m):
        hi = m.astype(np.dtype(jnp.bfloat16))
        lo = (m - hi.astype(np.float32)).astype(np.dtype(jnp.bfloat16))
        return jnp.asarray(hi), jnp.asarray(lo)

    return split(CF), split(SF), split(ICc), split(ICs)


def _proj_T_kernel(x_ref, w_ref, b_ref, o_ref):
    y = _bdot(x_ref[...], w_ref[...]) + b_ref[...]
    o_ref[...] = y.T


def _proj_to_lanes(x, W, b, B, L, tblk):
    M, K = x.shape
    N = W.shape[1]
    TB = L // tblk
    return pl.pallas_call(
        _proj_T_kernel,
        grid=(M // tblk,),
        in_specs=[pl.BlockSpec((tblk, K), lambda i: (i, 0)),
                  pl.BlockSpec((K, N), lambda i: (0, 0)),
                  pl.BlockSpec((1, N), lambda i: (0, 0))],
        out_specs=pl.BlockSpec((N, tblk), lambda i: (i // TB, i % TB)),
        out_shape=jax.ShapeDtypeStruct((B * N, L), jnp.float32),
        interpret=False,
    )(x, W, b.reshape(1, N))


def _mm_from_lanes_kernel(x_ref, w_ref, b_ref, o_ref):
    o_ref[...] = _bdot(x_ref[...].T, w_ref[...]) + b_ref[...]


def _mm_from_lanes(xt, W, b, B, L, tblk):
    D = W.shape[0]
    N = W.shape[1]
    TB = L // tblk
    return pl.pallas_call(
        _mm_from_lanes_kernel,
        grid=(B * TB,),
        in_specs=[pl.BlockSpec((D, tblk), lambda i: (i // TB, i % TB)),
                  pl.BlockSpec((D, N), lambda i: (0, 0)),
                  pl.BlockSpec((1, N), lambda i: (0, 0))],
        out_specs=pl.BlockSpec((tblk, N), lambda i: (i, 0)),
        out_shape=jax.ShapeDtypeStruct((B * L, N), jnp.float32),
        interpret=False,
    )(xt, W, b.reshape(1, N))


def _fwd2_kernel(qt_ref, kt_ref, cfh_ref, cfl_ref, sfh_ref, sfl_ref,
                 qr_ref, qi_ref, kr_ref, ki_ref):
    cfh, cfl = cfh_ref[...], cfl_ref[...]
    sfh, sfl = sfh_ref[...], sfl_ref[...]

    def fwd(x):
        xh, xl = _split_bf16(x)
        xr = _bdot(xh, cfh) + (_bdot(xh, cfl) + _bdot(xl, cfh))
        xi = _bdot(xh, sfh) + (_bdot(xh, sfl) + _bdot(xl, sfh))
        return xr, xi

    qr_ref[...], qi_ref[...] = fwd(qt_ref[...])
    kr_ref[...], ki_ref[...] = fwd(kt_ref[...])


def _fwd2(qt, kt, CFp, SFp, blk):
    M, L = qt.shape
    FP = CFp[0].shape[1]
    fmat = lambda: pl.BlockSpec((L, FP), lambda i: (0, 0))
    rowL = lambda: pl.BlockSpec((blk, L), lambda i: (i, 0))
    rowF = lambda: pl.BlockSpec((blk, FP), lambda i: (i, 0))
    o = jax.ShapeDtypeStruct((M, FP), jnp.float32)
    return pl.pallas_call(
        _fwd2_kernel,
        grid=(M // blk,),
        in_specs=[rowL(), rowL(), fmat(), fmat(), fmat(), fmat()],
        out_specs=[rowF(), rowF(), rowF(), rowF()],
        out_shape=[o, o, o, o],
        interpret=False,
    )(qt, kt, CFp[0], CFp[1], SFp[0], SFp[1])


def _rsel_kernel(qr_ref, qi_ref, kr_ref, ki_ref, icch_ref, iccl_ref,
                 icsh_ref, icsl_ref, c_ref, *, L, k):
    qr, qi = qr_ref[...], qi_ref[...]
    kr, ki = kr_ref[...], ki_ref[...]
    vals = (_dot3(qr * kr + qi * ki, icch_ref[...], iccl_ref[...])
            + _dot3(qr * ki - qi * kr, icsh_ref[...], icsl_ref[...]))
    iota = jax.lax.broadcasted_iota(jnp.int32, vals.shape, 1)
    ws, ds = [], []
    for _ in range(k):
        m = jnp.max(vals, axis=1, keepdims=True)
        idx = jnp.min(jnp.where(vals == m, iota, L), axis=1, keepdims=True)
        ws.append(m)
        ds.append(idx)
        vals = jnp.where(iota == idx, -jnp.inf, vals)
    w = jnp.concatenate(ws, axis=1)
    p = jax.nn.softmax(w, axis=1)
    c = jnp.zeros((vals.shape[0], L), jnp.float32)
    for i in range(k):
        c = c + jnp.where(iota == ds[i], p[:, i:i + 1], 0.0)
    c_ref[...] = c


def _rsel(Qr, Qi, Kr, Ki, ICcp, ICsp, kk, blk):
    M, FP = Qr.shape
    L = ICcp[0].shape[1]
    imat = lambda: pl.BlockSpec((FP, L), lambda i: (0, 0))
    rowF = lambda: pl.BlockSpec((blk, FP), lambda i: (i, 0))
    return pl.pallas_call(
        functools.partial(_rsel_kernel, L=L, k=kk),
        grid=(M // blk,),
        in_specs=[rowF(), rowF(), rowF(), rowF(),
                  imat(), imat(), imat(), imat()],
        out_specs=pl.BlockSpec((blk, L), lambda i: (i, 0)),
        out_shape=jax.ShapeDtypeStruct((M, L), jnp.float32),
        interpret=False,
    )(Qr, Qi, Kr, Ki, ICcp[0], ICcp[1], ICsp[0], ICsp[1])


def _recon_kernel(c_ref, qr_ref, qi_ref, cfh_ref, cfl_ref, sfh_ref, sfl_ref,
                  icch_ref, iccl_ref, icsh_ref, icsl_ref, agg_ref):
    ch, cl = _split_bf16(c_ref[...])
    cfh, cfl = cfh_ref[...], cfl_ref[...]
    sfh, sfl = sfh_ref[...], sfl_ref[...]
    cr = _bdot(ch, cfh) + (_bdot(ch, cfl) + _bdot(cl, cfh))
    ci = _bdot(ch, sfh) + (_bdot(ch, sfl) + _bdot(cl, sfh))
    qr, qi = qr_ref[...], qi_ref[...]
    agg_ref[...] = (_dot3(qr * cr + qi * ci, icch_ref[...], iccl_ref[...])
                    + _dot3(qr * ci - qi * cr, icsh_ref[...], icsl_ref[...]))


def _recon(c, Qr, Qi, CFp, SFp, ICcp, ICsp, blk):
    M, L = c.shape
    FP = CFp[0].shape[1]
    fmat = lambda: pl.BlockSpec((L, FP), lambda i: (0, 0))
    imat = lambda: pl.BlockSpec((FP, L), lambda i: (0, 0))
    rowL = lambda: pl.BlockSpec((blk, L), lambda i: (i, 0))
    rowF = lambda: pl.BlockSpec((blk, FP), lambda i: (i, 0))
    return pl.pallas_call(
        _recon_kernel,
        grid=(M // blk,),
        in_specs=[rowL(), rowF(), rowF(), fmat(), fmat(), fmat(), fmat(),
                  imat(), imat(), imat(), imat()],
        out_specs=rowL(),
        out_shape=jax.ShapeDtypeStruct((M, L), jnp.float32),
        interpret=False,
    )(c, Qr, Qi, CFp[0], CFp[1], SFp[0], SFp[1],
      ICcp[0], ICcp[1], ICsp[0], ICsp[1])


def _pipeline(q, k, Wq, bq, Wk, bk, Wo, bo):
    B, L, D = q.shape
    H = _H
    depth = D // H
    lanes = B * H * depth
    kk = int(2 * math.log(L))

    mm_blk = min(512, L)
    lane_blk = min(256, lanes)

    qt = _proj_to_lanes(q.reshape(B * L, D), Wq, bq, B, L, mm_blk)
    kt = _proj_to_lanes(k.reshape(B * L, D), Wk, bk, B, L, mm_blk)

    CFp, SFp, ICcp, ICsp = _dft_mats(L)
    Qr, Qi, Kr, Ki = _fwd2(qt, kt, CFp, SFp, lane_blk)
    c = _rsel(Qr, Qi, Kr, Ki, ICcp, ICsp, kk, lane_blk)
    agg = _recon(c, Qr, Qi, CFp, SFp, ICcp, ICsp, lane_blk)

    out = _mm_from_lanes(agg, Wo, bo, B, L, mm_blk)
    return out.reshape(B, L, D)


def kernel(q, k, v, Wq, bq, Wk, bk, Wv, bv, Wo, bo):
    B = q.shape[0]
    devs = jax.devices()
    nd = 2 if (len(devs) >= 2 and B % 2 == 0) else 1
    if nd == 1:
        return _pipeline(q, k, Wq, bq, Wk, bk, Wo, bo)
    from jax.experimental.shard_map import shard_map
    from jax.sharding import Mesh, PartitionSpec as P
    mesh = Mesh(np.array(devs[:nd]), ("x",))
    Bloc = B // nd

    def body(q, k, Wq, bq, Wk, bk, Wo, bo):
        i = jax.lax.axis_index("x")
        qloc = jax.lax.dynamic_slice_in_dim(q, i * Bloc, Bloc, 0)
        kloc = jax.lax.dynamic_slice_in_dim(k, i * Bloc, Bloc, 0)
        return _pipeline(qloc, kloc, Wq, bq, Wk, bk, Wo, bo)

    fn = shard_map(
        body, mesh=mesh,
        in_specs=(P(), P(), P(), P(), P(), P(), P(), P()),
        out_specs=P("x"), check_rep=False)
    return fn(q, k, Wq, bq, Wk, bk, Wo, bo)
